# baseline (speedup 1.0000x reference)
import functools

import jax
import jax.numpy as jnp
from jax import lax
from jax.experimental import pallas as pl
from jax.experimental.pallas import tpu as pltpu
from jax.experimental.pallas import tpu_sc as plsc

DIM = 32
PAD = 0
CHUNK = 320
SUPER = 128 // DIM


@functools.cache
def _build(B):
    info = plsc.get_sparse_core_info()
    nc, ns = info.num_cores, info.num_subcores
    nw = nc * ns
    per_w = B // nw
    assert per_w * nw == B and per_w % CHUNK == 0
    nchunk = per_w // CHUNK

    mesh = plsc.VectorSubcoreMesh(core_axis_name="c", subcore_axis_name="s")

    @functools.partial(
        pl.kernel,
        mesh=mesh,
        out_type=jax.ShapeDtypeStruct((B, DIM), jnp.float32),
        scratch_types=[
            pltpu.VMEM((2, CHUNK), jnp.int32),
            pltpu.VMEM((2, CHUNK), jnp.int32),
            pltpu.VMEM((2, CHUNK, 128), jnp.float32),
            pltpu.VMEM((2, CHUNK, DIM), jnp.float32),
            pltpu.SemaphoreType.DMA((2,)),
            pltpu.SemaphoreType.DMA,
        ],
        compiler_params=pltpu.CompilerParams(use_tc_tiling_on_sc=False),
    )
    def k(idx_hbm, table_hbm, out_hbm, idx_v, sidx_v, rows_v, sel_v, gsem, wsem):
        wid = lax.axis_index("s") * nc + lax.axis_index("c")
        base = wid * per_w

        def fire(ci, buf):
            pltpu.sync_copy(
                idx_hbm.at[pl.ds(base + ci * CHUNK, CHUNK)], idx_v.at[buf]
            )
            for l in range(CHUNK // 16):
                sidx_v[buf, pl.ds(l * 16, 16)] = (
                    idx_v[buf, pl.ds(l * 16, 16)] >> 2
                )
            pltpu.async_copy(
                table_hbm.at[sidx_v.at[buf]], rows_v.at[buf], gsem.at[buf]
            )

        fire(0, 0)

        def chunk_body(ci, carry):
            buf = ci & 1
            nbuf = 1 - buf

            @pl.when(ci >= 1)
            def _wait_write():
                pltpu.make_async_copy(
                    sel_v.at[nbuf], out_hbm.at[pl.ds(base, CHUNK)], wsem
                ).wait()

            @pl.when(ci + 1 < nchunk)
            def _fire_next():
                fire(ci + 1, nbuf)

            mm = idx_v[buf, pl.ds(0, 16)]
            for l in range(1, CHUNK // 16):
                mm = jnp.minimum(mm, idx_v[buf, pl.ds(l * 16, 16)])
            sm = mm[0]
            for i in range(1, 16):
                sm = jnp.minimum(sm, mm[i])

            pltpu.make_async_copy(
                table_hbm.at[sidx_v.at[buf]], rows_v.at[buf], gsem.at[buf]
            ).wait()

            def sel_grp(g, c2):
                vec = idx_v[buf, pl.ds(g * 16, 16)]
                for i in range(16):
                    r = g * 16 + i
                    sub = (vec[i] & (SUPER - 1)) * DIM
                    sel_v[buf, r, pl.ds(0, 16)] = rows_v[buf, r, pl.ds(sub, 16)]
                    sel_v[buf, r, pl.ds(16, 16)] = rows_v[
                        buf, r, pl.ds(sub + 16, 16)
                    ]
                return c2

            lax.fori_loop(0, CHUNK // 16, sel_grp, 0)

            zeros = jnp.zeros((16,), jnp.float32)
            ub = jnp.where(sm == PAD, CHUNK // 16, 0)

            def fix_grp(g, c2):
                vec = idx_v[buf, pl.ds(g * 16, 16)]
                for i in range(16):
                    @pl.when(vec[i] == PAD)
                    def _z():
                        r = g * 16 + i
                        sel_v[buf, r, pl.ds(0, 16)] = zeros
                        sel_v[buf, r, pl.ds(16, 16)] = zeros
                return c2

            lax.fori_loop(0, ub, fix_grp, 0)

            pltpu.async_copy(
                sel_v.at[buf], out_hbm.at[pl.ds(base + ci * CHUNK, CHUNK)], wsem
            )
            return carry

        lax.fori_loop(0, nchunk, chunk_body, 0)

        pltpu.make_async_copy(
            sel_v.at[0], out_hbm.at[pl.ds(base, CHUNK)], wsem
        ).wait()

    return k


def kernel(x, weight):
    b0, b1 = x.shape
    B = b0 * b1
    xf = x.reshape(B).astype(jnp.int32)
    w128 = weight.reshape(weight.shape[0] // SUPER, 128)
    out = _build(B)(xf, w128)
    return out.reshape(b0, b1, DIM)


# speedup vs baseline: 1.1339x; 1.1339x over previous
import functools

import jax
import jax.numpy as jnp
from jax import lax
from jax.experimental import pallas as pl
from jax.experimental.pallas import tpu as pltpu
from jax.experimental.pallas import tpu_sc as plsc

DIM = 32
PAD = 0
SUB = 128
CHUNK = 1024
NSUB = CHUNK // SUB


@functools.cache
def _build(B):
    info = plsc.get_sparse_core_info()
    nc, ns = info.num_cores, info.num_subcores
    nw = nc * ns
    per_w = B // nw
    assert per_w * nw == B and per_w % CHUNK == 0
    nchunk = per_w // CHUNK

    mesh = plsc.VectorSubcoreMesh(core_axis_name="c", subcore_axis_name="s")

    @functools.partial(
        pl.kernel,
        mesh=mesh,
        out_type=jax.ShapeDtypeStruct((B, DIM), jnp.float32),
        scratch_types=[
            pltpu.VMEM((2, NSUB, SUB), jnp.int32),
            pltpu.VMEM((2, CHUNK), jnp.int32),
            pltpu.VMEM((2, CHUNK, DIM), jnp.float32),
            pltpu.SemaphoreType.DMA((2,)),
            pltpu.SemaphoreType.DMA,
        ],
        compiler_params=pltpu.CompilerParams(use_tc_tiling_on_sc=False),
    )
    def k(idx_hbm, table_hbm, out_hbm, idx_v, fidx_v, rows_v, gsem, wsem):
        wid = lax.axis_index("s") * nc + lax.axis_index("c")
        base = wid * per_w
        irow0 = wid * (per_w // SUB)

        def fire(ci, buf):
            pltpu.sync_copy(
                idx_hbm.at[pl.ds(irow0 + ci * NSUB, NSUB)], idx_v.at[buf]
            )
            for j in range(NSUB):
                for l in range(SUB // 16):
                    v = idx_v[buf, j, pl.ds(l * 16, 16)]
                    f = ((v >> 3) << 5) + ((v & 7) << 2)
                    fidx_v[buf, pl.ds(j * SUB + l * 16, 16)] = f
            pltpu.async_copy(
                table_hbm.at[fidx_v.at[buf]], rows_v.at[buf], gsem.at[buf]
            )

        fire(0, 0)

        def chunk_body(ci, carry):
            buf = ci & 1
            nbuf = 1 - buf

            @pl.when(ci >= 1)
            def _wait_write():
                pltpu.make_async_copy(
                    rows_v.at[nbuf], out_hbm.at[pl.ds(base, CHUNK)], wsem
                ).wait()

            @pl.when(ci + 1 < nchunk)
            def _fire_next():
                fire(ci + 1, nbuf)

            mm = fidx_v[buf, pl.ds(0, 16)]
            for l in range(1, CHUNK // 16):
                mm = jnp.minimum(mm, fidx_v[buf, pl.ds(l * 16, 16)])
            sm = mm[0]
            for i in range(1, 16):
                sm = jnp.minimum(sm, mm[i])

            pltpu.make_async_copy(
                table_hbm.at[fidx_v.at[buf]], rows_v.at[buf], gsem.at[buf]
            ).wait()

            zeros = jnp.zeros((16,), jnp.float32)
            ub = jnp.where(sm == PAD, CHUNK // 16, 0)

            def fix_grp(g, c2):
                vec = fidx_v[buf, pl.ds(g * 16, 16)]
                for i in range(16):
                    @pl.when(vec[i] == PAD)
                    def _z():
                        r = g * 16 + i
                        rows_v[buf, r, pl.ds(0, 16)] = zeros
                        rows_v[buf, r, pl.ds(16, 16)] = zeros
                return c2

            lax.fori_loop(0, ub, fix_grp, 0)

            pltpu.async_copy(
                rows_v.at[buf], out_hbm.at[pl.ds(base + ci * CHUNK, CHUNK)], wsem
            )
            return carry

        lax.fori_loop(0, nchunk, chunk_body, 0)

        pltpu.make_async_copy(
            rows_v.at[0], out_hbm.at[pl.ds(base, CHUNK)], wsem
        ).wait()

    return k


def kernel(x, weight):
    b0, b1 = x.shape
    B = b0 * b1
    xf = x.reshape(B).astype(jnp.int32).reshape(B // SUB, SUB)
    w4 = jnp.pad(
        weight.reshape(-1, 8, DIM), ((0, 0), (0, 0), (0, 128 - DIM))
    ).reshape(-1, DIM)
    out = _build(B)(xf, w4)
    return out.reshape(b0, b1, DIM)
